# Initial kernel scaffold; baseline (speedup 1.0000x reference)
import jax
import jax.numpy as jnp
from jax.experimental import pallas as pl


def kernel(channel_emb, patch_emb, sel_W, sel_b, A_stack, C_stack, W_ih, W_hh, b_ih, b_hh):
    raise NotImplementedError("write your pallas kernel here")


# speedup vs baseline: 1.1695x; 1.1695x over previous
import functools

import jax
import jax.numpy as jnp
from jax import lax
from jax.experimental import pallas as pl
from jax.experimental.pallas import tpu as pltpu

NS = 5
B = 64
S = 3072
H = 1536
JT = 6
NT = S // JT
CT = H // JT


def _dot(a, b, dims):
    return lax.dot_general(a, b, dimension_numbers=(dims, ((), ())),
                           preferred_element_type=jnp.float32)


def _body(x_ref, selw_ref, selb_ref, a_ref, c_ref, w3_ref, bih_ref, bhh_ref,
          out_ref, acc_ref, idx_ref):
    s = pl.program_id(0)
    j = pl.program_id(1)

    @pl.when((s == 0) & (j == 0))
    def _router():
        x = x_ref[...]
        logits = _dot(x, selw_ref[...], ((1,), (1,))) + selb_ref[...]
        mx = jnp.max(logits, axis=1, keepdims=True)
        cols = lax.broadcasted_iota(jnp.int32, (B, NS), 1)
        idx = jnp.min(jnp.where(logits == mx, cols, NS), axis=1, keepdims=True)
        idx_ref[...] = jnp.broadcast_to(idx, (B, 128))
        acc_ref[...] = jnp.zeros((B, S), jnp.float32)

    @pl.when(s < NS)
    def _expert():
        x = x_ref[...]
        t = jnp.maximum(_dot(x, a_ref[0], ((1,), (0,))), 0.0)
        contrib = _dot(t, c_ref[0], ((1,), (0,)))
        mask = idx_ref[:, 0:1] == s
        acc_ref[...] += jnp.where(mask, contrib, 0.0)

    @pl.when(s == NS)
    def _gru():
        obs = acc_ref[...]
        gi = [_dot(obs, w3_ref[g], ((1,), (1,))) for g in range(3)]
        bih = [bih_ref[g:g + 1, pl.ds(j * CT, CT)] for g in range(3)]
        bhh = [bhh_ref[g:g + 1, pl.ds(j * CT, CT)] for g in range(3)]
        r = jax.nn.sigmoid(gi[0] + bih[0] + bhh[0])
        z = jax.nn.sigmoid(gi[1] + bih[1] + bhh[1])
        n = jnp.tanh(gi[2] + bih[2] + r * bhh[2])
        ch = x_ref[:, pl.ds(j * CT, CT)]
        pa = x_ref[:, pl.ds(H + j * CT, CT)]
        out_ref[...] = (1.0 - z) * n + ch + pa


@jax.jit
def _run(x, sel_W, sel_b2, A_stack, C_stack, W3, bih2, bhh2):
    grid = (NS + 1, JT)
    return pl.pallas_call(
        _body,
        grid=grid,
        in_specs=[
            pl.BlockSpec((B, S), lambda s, j: (0, 0)),
            pl.BlockSpec((NS, S), lambda s, j: (0, 0)),
            pl.BlockSpec((1, NS), lambda s, j: (0, 0)),
            pl.BlockSpec((1, S, NT),
                         lambda s, j: (jnp.minimum(s, NS - 1), 0,
                                       jnp.where(s < NS, j, JT - 1))),
            pl.BlockSpec((1, NT, S),
                         lambda s, j: (jnp.minimum(s, NS - 1),
                                       jnp.where(s < NS, j, JT - 1), 0)),
            pl.BlockSpec((3, CT, S),
                         lambda s, j: (0, jnp.where(s < NS, 0, j), 0)),
            pl.BlockSpec((3, H), lambda s, j: (0, 0)),
            pl.BlockSpec((3, H), lambda s, j: (0, 0)),
        ],
        out_specs=pl.BlockSpec((B, CT), lambda s, j: (0, jnp.where(s < NS, 0, j))),
        out_shape=jax.ShapeDtypeStruct((B, H), jnp.float32),
        scratch_shapes=[
            pltpu.VMEM((B, S), jnp.float32),
            pltpu.VMEM((B, 128), jnp.int32),
        ],
    )(x, sel_W, sel_b2, A_stack, C_stack, W3, bih2, bhh2)


def kernel(channel_emb, patch_emb, sel_W, sel_b, A_stack, C_stack, W_ih, W_hh,
           b_ih, b_hh):
    x = jnp.concatenate([channel_emb, patch_emb], axis=-1)
    return _run(x, sel_W, sel_b.reshape(1, NS), A_stack, C_stack,
                W_ih.reshape(3, H, S), b_ih.reshape(3, H), b_hh.reshape(3, H))
